# baseline (speedup 1.0000x reference)
import functools

import jax
import jax.numpy as jnp
from jax.experimental import pallas as pl
from jax.experimental.pallas import tpu as pltpu

_K = 7
_PAD = 3


def _sa_body(w_ref, b_ref, mask_ref, *refs, n_tile, n_split, Cs, C, W, HW,
             LPAD, inv_c):
    x_refs = refs[:n_split]
    o_ref = refs[n_split]
    padm_ref, pada_ref = refs[n_split + 1:]

    zpad = jnp.zeros((n_tile, LPAD), dtype=jnp.float32)
    padm_ref[:, :LPAD] = zpad
    padm_ref[:, LPAD + HW:] = zpad
    pada_ref[:, :LPAD] = zpad
    pada_ref[:, LPAD + HW:] = zpad

    for t in range(n_tile):
        acc_m = x_refs[0][t, 0:8, :]
        acc_s = acc_m
        first = True
        for xr in x_refs:
            for r in range(0, Cs - (Cs % 8), 8):
                if first:
                    first = False
                    continue
                blk = xr[t, r:r + 8, :]
                acc_m = jnp.maximum(acc_m, blk)
                acc_s = acc_s + blk
            if Cs % 8:
                blk = xr[t, Cs - (Cs % 8):Cs, :]
                acc_m = jnp.maximum(acc_m, jnp.max(blk, axis=0, keepdims=True))
                acc_s = acc_s + jnp.sum(blk, axis=0, keepdims=True)
        m = jnp.max(acc_m, axis=0, keepdims=True)
        s = jnp.sum(acc_s, axis=0, keepdims=True)
        padm_ref[t:t + 1, LPAD:LPAD + HW] = m
        pada_ref[t:t + 1, LPAD:LPAD + HW] = s * inv_c

    wv = [w_ref[i] for i in range(2 * _K * _K)]
    bv = b_ref[0]

    acc = jnp.zeros((n_tile, HW), dtype=jnp.float32)
    for dx in range(_K):
        pm = jnp.zeros((n_tile, HW), dtype=jnp.float32)
        pa = jnp.zeros((n_tile, HW), dtype=jnp.float32)
        for dy in range(_K):
            off = LPAD + (dy - _PAD) * W + (dx - _PAD)
            pm = pm + wv[dy * _K + dx] * padm_ref[:, off:off + HW]
            pa = pa + wv[_K * _K + dy * _K + dx] * pada_ref[:, off:off + HW]
        acc = acc + (pm + pa) * mask_ref[dx:dx + 1, :]

    o_ref[:, 0, :] = jax.nn.sigmoid(acc + bv).astype(o_ref.dtype)


def kernel(x, weight, bias):
    N, C, H, W = x.shape
    HW = H * W
    itemsize = jnp.dtype(x.dtype).itemsize

    n_tile = 1
    for t in (8, 4, 2):
        if N % t == 0:
            n_tile = t
            break
    n_split = 1
    for s in (8, 4, 2):
        if C % (8 * s) == 0:
            n_split = s
            break
    Cs = C // n_split

    LPAD = ((_PAD * (W + 1) + 127) // 128) * 128
    Wpad = 2 * LPAD + HW

    x_flat = x.reshape(N, C, HW)
    w_flat = weight.reshape(-1).astype(jnp.float32)
    b = bias.astype(jnp.float32)

    cols = jnp.tile(jnp.arange(W, dtype=jnp.int32), H)
    dxs = jnp.arange(_K, dtype=jnp.int32)[:, None]
    colmask = ((cols[None, :] + dxs - _PAD >= 0)
               & (cols[None, :] + dxs - _PAD < W)).astype(jnp.float32)

    body = functools.partial(_sa_body, n_tile=n_tile, n_split=n_split, Cs=Cs,
                             C=C, W=W, HW=HW, LPAD=LPAD, inv_c=1.0 / float(C))

    cost = pl.CostEstimate(
        flops=int(N * HW * (2 * C + 4 * _K * _K + _K)),
        transcendentals=int(N * HW),
        bytes_accessed=int(N * C * HW * itemsize + N * HW * itemsize
                           + _K * HW * 4 + (2 * _K * _K + 1) * 4),
    )

    x_specs = [
        pl.BlockSpec((n_tile, Cs, HW), functools.partial(
            lambda n, k: (n, k, 0), k=k))
        for k in range(n_split)
    ]

    out = pl.pallas_call(
        body,
        out_shape=jax.ShapeDtypeStruct((N, 1, HW), x.dtype),
        grid=(N // n_tile,),
        in_specs=[
            pl.BlockSpec(memory_space=pltpu.SMEM),
            pl.BlockSpec(memory_space=pltpu.SMEM),
            pl.BlockSpec((_K, HW), lambda n: (0, 0)),
            *x_specs,
        ],
        out_specs=pl.BlockSpec((n_tile, 1, HW), lambda n: (n, 0, 0)),
        scratch_shapes=[
            pltpu.VMEM((n_tile, Wpad), jnp.float32),
            pltpu.VMEM((n_tile, Wpad), jnp.float32),
        ],
        compiler_params=pltpu.CompilerParams(
            dimension_semantics=("parallel",)),
        cost_estimate=cost,
    )(w_flat, b, colmask, *([x_flat] * n_split))

    return out.reshape(N, 1, H, W)


# speedup vs baseline: 1.0206x; 1.0206x over previous
import functools

import jax
import jax.numpy as jnp
from jax.experimental import pallas as pl
from jax.experimental.pallas import tpu as pltpu

_K = 7
_PAD = 3


def _sa_body(w_ref, b_ref, mask_ref, *refs, n_tile, n_split, Cs, C, W, HW,
             LPAD, inv_c):
    x_refs = refs[:n_split]
    o_ref = refs[n_split]
    padm_ref, pada_ref = refs[n_split + 1:]

    zpad = jnp.zeros((n_tile, LPAD), dtype=jnp.float32)
    padm_ref[:, :LPAD] = zpad
    padm_ref[:, LPAD + HW:] = zpad
    pada_ref[:, :LPAD] = zpad
    pada_ref[:, LPAD + HW:] = zpad

    for t in range(n_tile):
        acc_m = x_refs[0][t, 0:8, :]
        acc_s = acc_m
        first = True
        for xr in x_refs:
            for r in range(0, Cs - (Cs % 8), 8):
                if first:
                    first = False
                    continue
                blk = xr[t, r:r + 8, :]
                acc_m = jnp.maximum(acc_m, blk)
                acc_s = acc_s + blk
            if Cs % 8:
                blk = xr[t, Cs - (Cs % 8):Cs, :]
                acc_m = jnp.maximum(acc_m, jnp.max(blk, axis=0, keepdims=True))
                acc_s = acc_s + jnp.sum(blk, axis=0, keepdims=True)
        m = jnp.max(acc_m, axis=0, keepdims=True)
        s = jnp.sum(acc_s, axis=0, keepdims=True)
        padm_ref[t:t + 1, LPAD:LPAD + HW] = m
        pada_ref[t:t + 1, LPAD:LPAD + HW] = s * inv_c

    wv = [w_ref[i] for i in range(2 * _K * _K)]
    bv = b_ref[0]

    acc = jnp.zeros((n_tile, HW), dtype=jnp.float32)
    for dx in range(_K):
        pm = jnp.zeros((n_tile, HW), dtype=jnp.float32)
        pa = jnp.zeros((n_tile, HW), dtype=jnp.float32)
        for dy in range(_K):
            off = LPAD + (dy - _PAD) * W + (dx - _PAD)
            pm = pm + wv[dy * _K + dx] * padm_ref[:, off:off + HW]
            pa = pa + wv[_K * _K + dy * _K + dx] * pada_ref[:, off:off + HW]
        acc = acc + (pm + pa) * mask_ref[dx:dx + 1, :]

    o_ref[:, 0, :] = jax.nn.sigmoid(acc + bv).astype(o_ref.dtype)


def kernel(x, weight, bias):
    N, C, H, W = x.shape
    HW = H * W
    itemsize = jnp.dtype(x.dtype).itemsize

    n_tile = 1
    for t in (16, 8, 4, 2):
        if N % t == 0:
            n_tile = t
            break
    n_split = 1
    for s in (8, 4, 2):
        if C % (8 * s) == 0:
            n_split = s
            break
    Cs = C // n_split

    LPAD = ((_PAD * (W + 1) + 127) // 128) * 128
    Wpad = 2 * LPAD + HW

    x_flat = x.reshape(N, C, HW)
    w_flat = weight.reshape(-1).astype(jnp.float32)
    b = bias.astype(jnp.float32)

    cols = jnp.tile(jnp.arange(W, dtype=jnp.int32), H)
    dxs = jnp.arange(_K, dtype=jnp.int32)[:, None]
    colmask = ((cols[None, :] + dxs - _PAD >= 0)
               & (cols[None, :] + dxs - _PAD < W)).astype(jnp.float32)

    body = functools.partial(_sa_body, n_tile=n_tile, n_split=n_split, Cs=Cs,
                             C=C, W=W, HW=HW, LPAD=LPAD, inv_c=1.0 / float(C))

    cost = pl.CostEstimate(
        flops=int(N * HW * (2 * C + 4 * _K * _K + _K)),
        transcendentals=int(N * HW),
        bytes_accessed=int(N * C * HW * itemsize + N * HW * itemsize
                           + _K * HW * 4 + (2 * _K * _K + 1) * 4),
    )

    x_specs = [
        pl.BlockSpec((n_tile, Cs, HW), functools.partial(
            lambda n, k: (n, k, 0), k=k))
        for k in range(n_split)
    ]

    out = pl.pallas_call(
        body,
        out_shape=jax.ShapeDtypeStruct((N, 1, HW), x.dtype),
        grid=(N // n_tile,),
        in_specs=[
            pl.BlockSpec(memory_space=pltpu.SMEM),
            pl.BlockSpec(memory_space=pltpu.SMEM),
            pl.BlockSpec((_K, HW), lambda n: (0, 0)),
            *x_specs,
        ],
        out_specs=pl.BlockSpec((n_tile, 1, HW), lambda n: (n, 0, 0)),
        scratch_shapes=[
            pltpu.VMEM((n_tile, Wpad), jnp.float32),
            pltpu.VMEM((n_tile, Wpad), jnp.float32),
        ],
        compiler_params=pltpu.CompilerParams(
            dimension_semantics=("parallel",)),
        cost_estimate=cost,
    )(w_flat, b, colmask, *([x_flat] * n_split))

    return out.reshape(N, 1, H, W)


# speedup vs baseline: 1.0208x; 1.0002x over previous
import functools

import jax
import jax.numpy as jnp
from jax.experimental import pallas as pl
from jax.experimental.pallas import tpu as pltpu

_K = 7
_PAD = 3


def _sa_body(w_ref, b_ref, mask_ref, *refs, n_tile, n_split, Cs, C, W, HW,
             LPAD, inv_c):
    x_refs = refs[:n_split]
    o_ref = refs[n_split]
    padm_ref, pada_ref = refs[n_split + 1:]

    zpad = jnp.zeros((n_tile, LPAD), dtype=jnp.float32)
    padm_ref[:, :LPAD] = zpad
    padm_ref[:, LPAD + HW:] = zpad
    pada_ref[:, :LPAD] = zpad
    pada_ref[:, LPAD + HW:] = zpad

    for t in range(n_tile):
        acc_m = x_refs[0][t, 0:8, :]
        acc_s = acc_m
        first = True
        for xr in x_refs:
            for r in range(0, Cs - (Cs % 8), 8):
                if first:
                    first = False
                    continue
                blk = xr[t, r:r + 8, :]
                acc_m = jnp.maximum(acc_m, blk)
                acc_s = acc_s + blk
            if Cs % 8:
                blk = xr[t, Cs - (Cs % 8):Cs, :]
                acc_m = jnp.maximum(acc_m, jnp.max(blk, axis=0, keepdims=True))
                acc_s = acc_s + jnp.sum(blk, axis=0, keepdims=True)
        m = jnp.max(acc_m, axis=0, keepdims=True)
        s = jnp.sum(acc_s, axis=0, keepdims=True)
        padm_ref[t:t + 1, LPAD:LPAD + HW] = m
        pada_ref[t:t + 1, LPAD:LPAD + HW] = s * inv_c

    wv = [w_ref[i] for i in range(2 * _K * _K)]
    bv = b_ref[0]

    acc = jnp.zeros((n_tile, HW), dtype=jnp.float32)
    for dx in range(_K):
        pm = jnp.zeros((n_tile, HW), dtype=jnp.float32)
        pa = jnp.zeros((n_tile, HW), dtype=jnp.float32)
        for dy in range(_K):
            off = LPAD + (dy - _PAD) * W + (dx - _PAD)
            pm = pm + wv[dy * _K + dx] * padm_ref[:, off:off + HW]
            pa = pa + wv[_K * _K + dy * _K + dx] * pada_ref[:, off:off + HW]
        acc = acc + (pm + pa) * mask_ref[dx:dx + 1, :]

    o_ref[:, 0, :] = jax.nn.sigmoid(acc + bv).astype(o_ref.dtype)


def kernel(x, weight, bias):
    N, C, H, W = x.shape
    HW = H * W
    itemsize = jnp.dtype(x.dtype).itemsize

    n_tile = 1
    for t in (16, 8, 4, 2):
        if N % t == 0:
            n_tile = t
            break
    n_split = 1
    Cs = C // n_split

    LPAD = ((_PAD * (W + 1) + 127) // 128) * 128
    Wpad = 2 * LPAD + HW

    x_flat = x.reshape(N, C, HW)
    w_flat = weight.reshape(-1).astype(jnp.float32)
    b = bias.astype(jnp.float32)

    cols = jnp.tile(jnp.arange(W, dtype=jnp.int32), H)
    dxs = jnp.arange(_K, dtype=jnp.int32)[:, None]
    colmask = ((cols[None, :] + dxs - _PAD >= 0)
               & (cols[None, :] + dxs - _PAD < W)).astype(jnp.float32)

    body = functools.partial(_sa_body, n_tile=n_tile, n_split=n_split, Cs=Cs,
                             C=C, W=W, HW=HW, LPAD=LPAD, inv_c=1.0 / float(C))

    cost = pl.CostEstimate(
        flops=int(N * HW * (2 * C + 4 * _K * _K + _K)),
        transcendentals=int(N * HW),
        bytes_accessed=int(N * C * HW * itemsize + N * HW * itemsize
                           + _K * HW * 4 + (2 * _K * _K + 1) * 4),
    )

    x_specs = [
        pl.BlockSpec((n_tile, Cs, HW), functools.partial(
            lambda n, k: (n, k, 0), k=k))
        for k in range(n_split)
    ]

    out = pl.pallas_call(
        body,
        out_shape=jax.ShapeDtypeStruct((N, 1, HW), x.dtype),
        grid=(N // n_tile,),
        in_specs=[
            pl.BlockSpec(memory_space=pltpu.SMEM),
            pl.BlockSpec(memory_space=pltpu.SMEM),
            pl.BlockSpec((_K, HW), lambda n: (0, 0)),
            *x_specs,
        ],
        out_specs=pl.BlockSpec((n_tile, 1, HW), lambda n: (n, 0, 0)),
        scratch_shapes=[
            pltpu.VMEM((n_tile, Wpad), jnp.float32),
            pltpu.VMEM((n_tile, Wpad), jnp.float32),
        ],
        compiler_params=pltpu.CompilerParams(
            dimension_semantics=("parallel",)),
        cost_estimate=cost,
    )(w_flat, b, colmask, *([x_flat] * n_split))

    return out.reshape(N, 1, H, W)
